# baseline (speedup 1.0000x reference)
import jax
import jax.numpy as jnp
from jax import lax
from jax.experimental import pallas as pl
from jax.experimental.pallas import tpu as pltpu
from jax.experimental.pallas import tpu_sc as plsc

N = 10000
D = 256
E = 160000
CW = 16

NC = 2
NS = 16
HALF = N // NC
HALF_PAD = 5120
DUMMY = 5100
EPT = E // NS
K = 80
NB = EPT // K
ZROWS = HALF_PAD // NS

BM = 400
GRID = N // BM


def _layer_norm(h, g, b):
    mu = jnp.mean(h, axis=1, keepdims=True)
    var = jnp.mean((h - mu) ** 2, axis=1, keepdims=True)
    return (h - mu) * lax.rsqrt(var + 1e-5) * g + b


def _gelu(h):
    return 0.5 * h * (1.0 + lax.erf(h * 0.7071067811865476))


def _dotT(a, w):
    return lax.dot_general(a, w, (((1,), (1,)), ((), ())),
                           preferred_element_type=jnp.float32)


def _tc1_body(x_ref, wd_ref, bd_ref, g1_ref, be1_ref, out_ref):
    h = _dotT(x_ref[...], wd_ref[...]) + bd_ref[...]
    out_ref[...] = _gelu(_layer_norm(h, g1_ref[...], be1_ref[...]))


def _tc2_body(s_ref, c_ref, h_ref, wl_ref, bl_ref, wr_ref, g_ref, be_ref,
              out_ref):
    inv = 1.0 / jnp.maximum(c_ref[:, 0:1], 1.0)
    mean = s_ref[...] * inv
    a = _dotT(mean, wl_ref[...]) + bl_ref[...] + _dotT(h_ref[...], wr_ref[...])
    out_ref[...] = _gelu(_layer_norm(a, g_ref[...], be_ref[...]))


def _tc3_body(s_ref, c_ref, h_ref, x_ref, wsc_ref, bsc_ref, wl_ref, bl_ref,
              wr_ref, out_ref):
    inv = 1.0 / jnp.maximum(c_ref[:, 0:1], 1.0)
    mean = s_ref[...] * inv
    out = _dotT(mean, wl_ref[...]) + bl_ref[...] + _dotT(h_ref[...], wr_ref[...])
    out_ref[...] = out + _dotT(x_ref[...], wsc_ref[...]) + bsc_ref[...]


def _row_spec(width):
    return pl.BlockSpec((BM, width), lambda i: (i, 0))


def _full_spec(shape):
    return pl.BlockSpec(shape, lambda i: tuple(0 for _ in shape))


_W = _full_spec((D, D))
_B = _full_spec((1, D))


def _tc1(x, wd, bd, g1, be1):
    return pl.pallas_call(
        _tc1_body,
        grid=(GRID,),
        in_specs=[_row_spec(D), _W, _B, _B, _B],
        out_specs=_row_spec(D),
        out_shape=jax.ShapeDtypeStruct((N, D), jnp.float32),
    )(x, wd, bd, g1, be1)


def _tc2(s, c, h, wl, bl, wr, g, be):
    return pl.pallas_call(
        _tc2_body,
        grid=(GRID,),
        in_specs=[_row_spec(D), _row_spec(CW), _row_spec(D), _W, _B, _W, _B,
                  _B],
        out_specs=_row_spec(D),
        out_shape=jax.ShapeDtypeStruct((N, D), jnp.float32),
    )(s, c, h, wl, bl, wr, g, be)


def _tc3(s, c, h, x, wsc, bsc, wl, bl, wr):
    return pl.pallas_call(
        _tc3_body,
        grid=(GRID,),
        in_specs=[_row_spec(D), _row_spec(CW), _row_spec(D), _row_spec(D),
                  _W, _B, _W, _B, _W],
        out_specs=_row_spec(D),
        out_shape=jax.ShapeDtypeStruct((N, D), jnp.float32),
    )(s, c, h, x, wsc, bsc, wl, bl, wr)


NW = NC * NS
OB = HALF_PAD
OUTP = NC * OB
NV = 2 * NW
TRV = OUTP // NV
SCRAP = TRV
AROWS = TRV + 8
LCAP = 6400
LIMIT = LCAP - K
ECH = 10000
VPB = K // 16


def _sc_mesh():
    return plsc.VectorSubcoreMesh(core_axis_name="c", subcore_axis_name="s")


def _sc_filter_body(src_hbm, dst_hbm, lsrc_hbm, lrow_hbm, lcnt_hbm,
                    esrc_v, edst_v, msrc0_v, mrow0_v, msrc1_v, mrow1_v,
                    cnt_v):
    cid = lax.axis_index("c")
    sid = lax.axis_index("s")
    wid = cid * NS + sid
    lo = wid * 2 * TRV

    def run_chunk(c, carry):
        pltpu.sync_copy(src_hbm.at[pl.ds(c * ECH, ECH)], esrc_v)
        pltpu.sync_copy(dst_hbm.at[pl.ds(c * ECH, ECH)], edst_v)

        def fstep(i, carry):
            off0, off1 = carry
            s = esrc_v[pl.ds(i * 16, 16)]
            d = edst_v[pl.ds(i * 16, 16)]
            p = d + jnp.where(d >= HALF, OB - HALF, 0)
            q = p - lo
            m0 = (q >= 0) & (q < TRV)
            m1 = (q >= TRV) & (q < 2 * TRV)
            plsc.store_compressed(msrc0_v.at[pl.ds(off0, 16)], s, mask=m0)
            plsc.store_compressed(mrow0_v.at[pl.ds(off0, 16)], q, mask=m0)
            plsc.store_compressed(msrc1_v.at[pl.ds(off1, 16)], s, mask=m1)
            plsc.store_compressed(mrow1_v.at[pl.ds(off1, 16)], q - TRV,
                                  mask=m1)
            off0 = jnp.minimum(
                off0 + jnp.max(plsc.all_reduce_population_count(m0)), LIMIT)
            off1 = jnp.minimum(
                off1 + jnp.max(plsc.all_reduce_population_count(m1)), LIMIT)
            return off0, off1

        return plsc.parallel_loop(0, ECH // 16, carry=carry)(fstep)

    cnt0 = jnp.int32(0)
    cnt1 = jnp.int32(0)
    for c in range(E // ECH):
        cnt0, cnt1 = run_chunk(c, (cnt0, cnt1))

    zi16 = jnp.zeros((16,), jnp.int32)
    for j in range(VPB):
        msrc0_v[pl.ds(cnt0 + j * 16, 16)] = zi16 + sid * 625
        mrow0_v[pl.ds(cnt0 + j * 16, 16)] = zi16 + SCRAP
        msrc1_v[pl.ds(cnt1 + j * 16, 16)] = zi16 + sid * 625
        mrow1_v[pl.ds(cnt1 + j * 16, 16)] = zi16 + SCRAP

    cnt_v[pl.ds(0, 16)] = zi16 + cnt0
    cnt_v[pl.ds(16, 16)] = zi16 + cnt1
    v0 = 2 * wid
    pltpu.sync_copy(msrc0_v, lsrc_hbm.at[pl.ds(v0 * LCAP, LCAP)])
    pltpu.sync_copy(mrow0_v, lrow_hbm.at[pl.ds(v0 * LCAP, LCAP)])
    pltpu.sync_copy(msrc1_v, lsrc_hbm.at[pl.ds((v0 + 1) * LCAP, LCAP)])
    pltpu.sync_copy(mrow1_v, lrow_hbm.at[pl.ds((v0 + 1) * LCAP, LCAP)])
    pltpu.sync_copy(cnt_v, lcnt_hbm.at[pl.ds(wid * 32, 32)])


def _sc_filter(src, dst):
    return pl.kernel(
        _sc_filter_body,
        out_type=(
            jax.ShapeDtypeStruct((NV * LCAP,), jnp.int32),
            jax.ShapeDtypeStruct((NV * LCAP,), jnp.int32),
            jax.ShapeDtypeStruct((NW * 32,), jnp.int32),
        ),
        mesh=_sc_mesh(),
        compiler_params=pltpu.CompilerParams(needs_layout_passes=False),
        scratch_types=[
            pltpu.VMEM((ECH,), jnp.int32),
            pltpu.VMEM((ECH,), jnp.int32),
            pltpu.VMEM((LCAP,), jnp.int32),
            pltpu.VMEM((LCAP,), jnp.int32),
            pltpu.VMEM((LCAP,), jnp.int32),
            pltpu.VMEM((LCAP,), jnp.int32),
            pltpu.VMEM((32,), jnp.int32),
        ],
    )(src, dst)


def _sc_agg_body(h_hbm, lsrc_hbm, lrow_hbm, lcnt_hbm, sums_hbm, cnts_hbm,
                 sgidx_v, lsv, lrv, cidx_v, grows_v, acc_s, acc_c,
                 sema, semb):
    cid = lax.axis_index("c")
    sid = lax.axis_index("s")
    wid = cid * NS + sid

    zeros16 = jnp.zeros((16,), jnp.float32)
    ones16 = jnp.ones((16,), jnp.float32)
    pltpu.sync_copy(lcnt_hbm.at[pl.ds(wid * 32, 32)], cidx_v)

    for ph in range(2):
        vid = 2 * wid + ph

        def zrow(i, carry):
            for j in range(D // 16):
                acc_s[i, pl.ds(j * 16, 16)] = zeros16
            acc_c[i, pl.ds(0, 16)] = zeros16
            return carry

        lax.fori_loop(0, AROWS, zrow, 0)

        pltpu.sync_copy(lsrc_hbm.at[pl.ds(vid * LCAP, LCAP)], lsv)
        pltpu.sync_copy(lrow_hbm.at[pl.ds(vid * LCAP, LCAP)], lrv)

        cnt = cidx_v[pl.ds(ph * 16, 16)][0]
        nb = (cnt + (K - 1)) // K

        def start_slot(b, s):
            base = s * K
            for j in range(VPB):
                sgidx_v[pl.ds(base + j * 16, 16)] = (
                    lsv[pl.ds(b * K + j * 16, 16)])
            sem = sema if s == 0 else semb
            pltpu.async_copy(h_hbm.at[sgidx_v.at[pl.ds(base, K)]],
                             grows_v.at[pl.ds(base, K)], sem)

        def wait_slot(s):
            base = s * K
            sem = sema if s == 0 else semb
            pltpu.make_async_copy(h_hbm.at[sgidx_v.at[pl.ds(base, K)]],
                                  grows_v.at[pl.ds(base, K)], sem).wait()

        @pl.when(nb > 0)
        def _():
            start_slot(0, 0)

        def batch(b, carry):
            slot = lax.rem(b, 2)
            more = b + 1 < nb

            @pl.when(more & (slot == 0))
            def _():
                start_slot(b + 1, 1)

            @pl.when(more & (slot == 1))
            def _():
                start_slot(b + 1, 0)

            @pl.when(slot == 0)
            def _():
                wait_slot(0)

            @pl.when(slot == 1)
            def _():
                wait_slot(1)

            gbase = slot * K

            @plsc.parallel_loop(0, VPB, unroll=2)
            def blk(q):
                rv = lrv[pl.ds(b * K + q * 16, 16)]
                for l in range(16):
                    row = rv[l]
                    for j in range(D // 16):
                        plsc.addupdate(acc_s.at[row, pl.ds(j * 16, 16)],
                                       grows_v[gbase + q * 16 + l,
                                               pl.ds(j * 16, 16)])
                    plsc.addupdate(acc_c.at[row, pl.ds(0, 16)], ones16)

            return carry

        lax.fori_loop(0, nb, batch, 0)

        pltpu.sync_copy(acc_s.at[pl.ds(0, TRV)],
                        sums_hbm.at[pl.ds(vid * TRV, TRV)])
        pltpu.sync_copy(acc_c.at[pl.ds(0, TRV)],
                        cnts_hbm.at[pl.ds(vid * TRV, TRV)])


def _sc_agg(h, lsrc, lrow, lcnt):
    return pl.kernel(
        _sc_agg_body,
        out_type=(
            jax.ShapeDtypeStruct((OUTP, D), jnp.float32),
            jax.ShapeDtypeStruct((OUTP, CW), jnp.float32),
        ),
        mesh=_sc_mesh(),
        scratch_types=[
            pltpu.VMEM((2 * K,), jnp.int32),
            pltpu.VMEM((LCAP,), jnp.int32),
            pltpu.VMEM((LCAP,), jnp.int32),
            pltpu.VMEM((32,), jnp.int32),
            pltpu.VMEM((2 * K, D), jnp.float32),
            pltpu.VMEM((AROWS, D), jnp.float32),
            pltpu.VMEM((AROWS, CW), jnp.float32),
            pltpu.SemaphoreType.DMA,
            pltpu.SemaphoreType.DMA,
        ],
    )(h, lsrc, lrow, lcnt)


def _unpad(a):
    return a.reshape(NC, OB, a.shape[-1])[:, :HALF].reshape(N, a.shape[-1])


def kernel(x, edges, W_down, b_down, W_sc, b_sc, g1, be1, Wl1, bl1, Wr1,
           g2, be2, Wl2, bl2, Wr2):
    src = edges[0]
    dst = edges[1]
    r = lambda v: v.reshape(1, D)
    lsrc, lrow, lcnt = _sc_filter(src, dst)
    h0 = _tc1(x, W_down, r(b_down), r(g1), r(be1))
    s1f, cntsf = _sc_agg(h0, lsrc, lrow, lcnt)
    s1, cnts = _unpad(s1f), _unpad(cntsf)
    g1p = _tc2(s1, cnts, h0, Wl1, r(bl1), Wr1, r(g2), r(be2))
    s2f, _ = _sc_agg(g1p, lsrc, lrow, lcnt)
    s2 = _unpad(s2f)
    return _tc3(s2, cnts, g1p, x, W_sc, r(b_sc), Wl2, r(bl2), Wr2)


# speedup vs baseline: 1.1562x; 1.1562x over previous
import jax
import jax.numpy as jnp
from jax import lax
from jax.experimental import pallas as pl
from jax.experimental.pallas import tpu as pltpu
from jax.experimental.pallas import tpu_sc as plsc

N = 10000
D = 256
E = 160000
CW = 16

NC = 2
NS = 16
HALF = N // NC
HALF_PAD = 5120
DUMMY = 5100
EPT = E // NS
K = 80
NB = EPT // K
ZROWS = HALF_PAD // NS

BM = 400
GRID = N // BM


def _layer_norm(h, g, b):
    mu = jnp.mean(h, axis=1, keepdims=True)
    var = jnp.mean((h - mu) ** 2, axis=1, keepdims=True)
    return (h - mu) * lax.rsqrt(var + 1e-5) * g + b


def _gelu(h):
    return 0.5 * h * (1.0 + lax.erf(h * 0.7071067811865476))


def _dotT(a, w):
    return lax.dot_general(a, w, (((1,), (1,)), ((), ())),
                           preferred_element_type=jnp.float32)


def _tc1_body(x_ref, wd_ref, bd_ref, g1_ref, be1_ref, out_ref):
    h = _dotT(x_ref[...], wd_ref[...]) + bd_ref[...]
    out_ref[...] = _gelu(_layer_norm(h, g1_ref[...], be1_ref[...]))


def _tc2_body(s_ref, c_ref, h_ref, wl_ref, bl_ref, wr_ref, g_ref, be_ref,
              out_ref):
    inv = 1.0 / jnp.maximum(c_ref[:, 0:1], 1.0)
    mean = s_ref[...] * inv
    a = _dotT(mean, wl_ref[...]) + bl_ref[...] + _dotT(h_ref[...], wr_ref[...])
    out_ref[...] = _gelu(_layer_norm(a, g_ref[...], be_ref[...]))


def _tc3_body(s_ref, c_ref, h_ref, x_ref, wsc_ref, bsc_ref, wl_ref, bl_ref,
              wr_ref, out_ref):
    inv = 1.0 / jnp.maximum(c_ref[:, 0:1], 1.0)
    mean = s_ref[...] * inv
    out = _dotT(mean, wl_ref[...]) + bl_ref[...] + _dotT(h_ref[...], wr_ref[...])
    out_ref[...] = out + _dotT(x_ref[...], wsc_ref[...]) + bsc_ref[...]


def _row_spec(width):
    return pl.BlockSpec((BM, width), lambda i: (i, 0))


def _full_spec(shape):
    return pl.BlockSpec(shape, lambda i: tuple(0 for _ in shape))


_W = _full_spec((D, D))
_B = _full_spec((1, D))


def _tc1(x, wd, bd, g1, be1):
    return pl.pallas_call(
        _tc1_body,
        grid=(GRID,),
        in_specs=[_row_spec(D), _W, _B, _B, _B],
        out_specs=_row_spec(D),
        out_shape=jax.ShapeDtypeStruct((N, D), jnp.float32),
    )(x, wd, bd, g1, be1)


def _tc2(s, c, h, wl, bl, wr, g, be):
    return pl.pallas_call(
        _tc2_body,
        grid=(GRID,),
        in_specs=[_row_spec(D), _row_spec(CW), _row_spec(D), _W, _B, _W, _B,
                  _B],
        out_specs=_row_spec(D),
        out_shape=jax.ShapeDtypeStruct((N, D), jnp.float32),
    )(s, c, h, wl, bl, wr, g, be)


def _tc3(s, c, h, x, wsc, bsc, wl, bl, wr):
    return pl.pallas_call(
        _tc3_body,
        grid=(GRID,),
        in_specs=[_row_spec(D), _row_spec(CW), _row_spec(D), _row_spec(D),
                  _W, _B, _W, _B, _W],
        out_specs=_row_spec(D),
        out_shape=jax.ShapeDtypeStruct((N, D), jnp.float32),
    )(s, c, h, x, wsc, bsc, wl, bl, wr)


NW = NC * NS
OB = HALF_PAD
OUTP = NC * OB
NV = 2 * NW
TRV = OUTP // NV
SCRAP = TRV
AROWS = TRV + 8
LCAP = 6400
LIMIT = LCAP - K
ECH = 10000
VPB = K // 16


def _sc_mesh():
    return plsc.VectorSubcoreMesh(core_axis_name="c", subcore_axis_name="s")


def _sc_filter_body(src_hbm, dst_hbm, lsrc_hbm, lrow_hbm, lcnt_hbm,
                    esrc_v, edst_v, msrc0_v, mrow0_v, msrc1_v, mrow1_v,
                    cnt_v):
    cid = lax.axis_index("c")
    sid = lax.axis_index("s")
    wid = cid * NS + sid
    lo = wid * 2 * TRV

    def run_chunk(c, carry):
        pltpu.sync_copy(src_hbm.at[pl.ds(c * ECH, ECH)], esrc_v)
        pltpu.sync_copy(dst_hbm.at[pl.ds(c * ECH, ECH)], edst_v)

        def fstep(i, carry):
            off0, off1 = carry
            s = esrc_v[pl.ds(i * 16, 16)]
            d = edst_v[pl.ds(i * 16, 16)]
            p = d + jnp.where(d >= HALF, OB - HALF, 0)
            q = p - lo
            m0 = (q >= 0) & (q < TRV)
            m1 = (q >= TRV) & (q < 2 * TRV)
            plsc.store_compressed(msrc0_v.at[pl.ds(off0, 16)], s, mask=m0)
            plsc.store_compressed(mrow0_v.at[pl.ds(off0, 16)], q, mask=m0)
            plsc.store_compressed(msrc1_v.at[pl.ds(off1, 16)], s, mask=m1)
            plsc.store_compressed(mrow1_v.at[pl.ds(off1, 16)], q - TRV,
                                  mask=m1)
            off0 = jnp.minimum(
                off0 + jnp.max(plsc.all_reduce_population_count(m0)), LIMIT)
            off1 = jnp.minimum(
                off1 + jnp.max(plsc.all_reduce_population_count(m1)), LIMIT)
            return off0, off1

        return plsc.parallel_loop(0, ECH // 16, carry=carry)(fstep)

    cnt0 = jnp.int32(0)
    cnt1 = jnp.int32(0)
    for c in range(E // ECH):
        cnt0, cnt1 = run_chunk(c, (cnt0, cnt1))

    zi16 = jnp.zeros((16,), jnp.int32)
    for j in range(VPB):
        msrc0_v[pl.ds(cnt0 + j * 16, 16)] = zi16 + sid * 625
        mrow0_v[pl.ds(cnt0 + j * 16, 16)] = zi16 + SCRAP
        msrc1_v[pl.ds(cnt1 + j * 16, 16)] = zi16 + sid * 625
        mrow1_v[pl.ds(cnt1 + j * 16, 16)] = zi16 + SCRAP

    cnt_v[pl.ds(0, 16)] = zi16 + cnt0
    cnt_v[pl.ds(16, 16)] = zi16 + cnt1
    v0 = 2 * wid
    pltpu.sync_copy(msrc0_v, lsrc_hbm.at[pl.ds(v0 * LCAP, LCAP)])
    pltpu.sync_copy(mrow0_v, lrow_hbm.at[pl.ds(v0 * LCAP, LCAP)])
    pltpu.sync_copy(msrc1_v, lsrc_hbm.at[pl.ds((v0 + 1) * LCAP, LCAP)])
    pltpu.sync_copy(mrow1_v, lrow_hbm.at[pl.ds((v0 + 1) * LCAP, LCAP)])
    pltpu.sync_copy(cnt_v, lcnt_hbm.at[pl.ds(wid * 32, 32)])


def _sc_filter(src, dst):
    return pl.kernel(
        _sc_filter_body,
        out_type=(
            jax.ShapeDtypeStruct((NV * LCAP,), jnp.int32),
            jax.ShapeDtypeStruct((NV * LCAP,), jnp.int32),
            jax.ShapeDtypeStruct((NW * 32,), jnp.int32),
        ),
        mesh=_sc_mesh(),
        compiler_params=pltpu.CompilerParams(needs_layout_passes=False),
        scratch_types=[
            pltpu.VMEM((ECH,), jnp.int32),
            pltpu.VMEM((ECH,), jnp.int32),
            pltpu.VMEM((LCAP,), jnp.int32),
            pltpu.VMEM((LCAP,), jnp.int32),
            pltpu.VMEM((LCAP,), jnp.int32),
            pltpu.VMEM((LCAP,), jnp.int32),
            pltpu.VMEM((32,), jnp.int32),
        ],
    )(src, dst)


def _sc_agg_body(h_hbm, lsrc_hbm, lrow_hbm, lcnt_hbm, sums_hbm, cnts_hbm,
                 sgidx_v, lsv, lrv, cidx_v, grows_v, acc_s, acc_c,
                 sema, semb):
    cid = lax.axis_index("c")
    sid = lax.axis_index("s")
    wid = cid * NS + sid

    zeros16 = jnp.zeros((16,), jnp.float32)
    ones16 = jnp.ones((16,), jnp.float32)
    pltpu.sync_copy(lcnt_hbm.at[pl.ds(wid * 32, 32)], cidx_v)

    for ph in range(2):
        vid = 2 * wid + ph

        def zrow(i, carry):
            for j in range(D // 16):
                acc_s[i, pl.ds(j * 16, 16)] = zeros16
            acc_c[i, pl.ds(0, 16)] = zeros16
            return carry

        lax.fori_loop(0, AROWS, zrow, 0)

        pltpu.sync_copy(lsrc_hbm.at[pl.ds(vid * LCAP, LCAP)], lsv)
        pltpu.sync_copy(lrow_hbm.at[pl.ds(vid * LCAP, LCAP)], lrv)

        cnt = cidx_v[pl.ds(ph * 16, 16)][0]
        nb = (cnt + (K - 1)) // K

        def start_slot(b, s):
            base = s * K
            for j in range(VPB):
                sgidx_v[pl.ds(base + j * 16, 16)] = (
                    lsv[pl.ds(b * K + j * 16, 16)])
            sem = sema if s == 0 else semb
            pltpu.async_copy(h_hbm.at[sgidx_v.at[pl.ds(base, K)]],
                             grows_v.at[pl.ds(base, K)], sem)

        def wait_slot(s):
            base = s * K
            sem = sema if s == 0 else semb
            pltpu.make_async_copy(h_hbm.at[sgidx_v.at[pl.ds(base, K)]],
                                  grows_v.at[pl.ds(base, K)], sem).wait()

        @pl.when(nb > 0)
        def _():
            start_slot(0, 0)

        def batch(b, carry):
            slot = lax.rem(b, 2)
            more = b + 1 < nb

            @pl.when(more & (slot == 0))
            def _():
                start_slot(b + 1, 1)

            @pl.when(more & (slot == 1))
            def _():
                start_slot(b + 1, 0)

            @pl.when(slot == 0)
            def _():
                wait_slot(0)

            @pl.when(slot == 1)
            def _():
                wait_slot(1)

            gbase = slot * K

            @plsc.parallel_loop(0, VPB)
            def blk(q):
                rv = lrv[pl.ds(b * K + q * 16, 16)]
                for l in range(16):
                    row = rv[l]
                    for j in range(D // 16):
                        plsc.addupdate(acc_s.at[row, pl.ds(j * 16, 16)],
                                       grows_v[gbase + q * 16 + l,
                                               pl.ds(j * 16, 16)])
                    plsc.addupdate(acc_c.at[row, pl.ds(0, 16)], ones16)

            return carry

        lax.fori_loop(0, nb, batch, 0)

        pltpu.sync_copy(acc_s.at[pl.ds(0, TRV)],
                        sums_hbm.at[pl.ds(vid * TRV, TRV)])
        pltpu.sync_copy(acc_c.at[pl.ds(0, TRV)],
                        cnts_hbm.at[pl.ds(vid * TRV, TRV)])


def _sc_agg(h, lsrc, lrow, lcnt):
    return pl.kernel(
        _sc_agg_body,
        out_type=(
            jax.ShapeDtypeStruct((OUTP, D), jnp.float32),
            jax.ShapeDtypeStruct((OUTP, CW), jnp.float32),
        ),
        mesh=_sc_mesh(),
        scratch_types=[
            pltpu.VMEM((2 * K,), jnp.int32),
            pltpu.VMEM((LCAP,), jnp.int32),
            pltpu.VMEM((LCAP,), jnp.int32),
            pltpu.VMEM((32,), jnp.int32),
            pltpu.VMEM((2 * K, D), jnp.float32),
            pltpu.VMEM((AROWS, D), jnp.float32),
            pltpu.VMEM((AROWS, CW), jnp.float32),
            pltpu.SemaphoreType.DMA,
            pltpu.SemaphoreType.DMA,
        ],
    )(h, lsrc, lrow, lcnt)


def _unpad(a):
    return a.reshape(NC, OB, a.shape[-1])[:, :HALF].reshape(N, a.shape[-1])


def kernel(x, edges, W_down, b_down, W_sc, b_sc, g1, be1, Wl1, bl1, Wr1,
           g2, be2, Wl2, bl2, Wr2):
    src = edges[0]
    dst = edges[1]
    r = lambda v: v.reshape(1, D)
    lsrc, lrow, lcnt = _sc_filter(src, dst)
    h0 = _tc1(x, W_down, r(b_down), r(g1), r(be1))
    s1f, cntsf = _sc_agg(h0, lsrc, lrow, lcnt)
    s1, cnts = _unpad(s1f), _unpad(cntsf)
    g1p = _tc2(s1, cnts, h0, Wl1, r(bl1), Wr1, r(g2), r(be2))
    s2f, _ = _sc_agg(g1p, lsrc, lrow, lcnt)
    s2 = _unpad(s2f)
    return _tc3(s2, cnts, g1p, x, W_sc, r(b_sc), Wl2, r(bl2), Wr2)


# speedup vs baseline: 1.1640x; 1.0068x over previous
import jax
import jax.numpy as jnp
from jax import lax
from jax.experimental import pallas as pl
from jax.experimental.pallas import tpu as pltpu
from jax.experimental.pallas import tpu_sc as plsc

N = 10000
D = 256
E = 160000
CW = 16

NC = 2
NS = 16
HALF = N // NC
HALF_PAD = 5120
K = 80

BM = 400
GRID = N // BM


def _layer_norm(h, g, b):
    mu = jnp.mean(h, axis=1, keepdims=True)
    var = jnp.mean((h - mu) ** 2, axis=1, keepdims=True)
    return (h - mu) * lax.rsqrt(var + 1e-5) * g + b


def _gelu(h):
    return 0.5 * h * (1.0 + lax.erf(h * 0.7071067811865476))


def _dotT(a, w):
    return lax.dot_general(a, w, (((1,), (1,)), ((), ())),
                           preferred_element_type=jnp.float32)


def _tc1_body(x_ref, wd_ref, bd_ref, g1_ref, be1_ref, out_ref):
    h = _dotT(x_ref[...], wd_ref[...]) + bd_ref[...]
    out_ref[...] = _gelu(_layer_norm(h, g1_ref[...], be1_ref[...]))


def _tc2_body(s_ref, c_ref, h_ref, wl_ref, bl_ref, wr_ref, g_ref, be_ref,
              out_ref):
    inv = 1.0 / jnp.maximum(c_ref[:, 0:1], 1.0)
    mean = s_ref[...] * inv
    a = _dotT(mean, wl_ref[...]) + bl_ref[...] + _dotT(h_ref[...], wr_ref[...])
    out_ref[...] = _gelu(_layer_norm(a, g_ref[...], be_ref[...]))


def _tc3_body(s_ref, c_ref, h_ref, x_ref, wsc_ref, bsc_ref, wl_ref, bl_ref,
              wr_ref, out_ref):
    inv = 1.0 / jnp.maximum(c_ref[:, 0:1], 1.0)
    mean = s_ref[...] * inv
    out = _dotT(mean, wl_ref[...]) + bl_ref[...] + _dotT(h_ref[...], wr_ref[...])
    out_ref[...] = out + _dotT(x_ref[...], wsc_ref[...]) + bsc_ref[...]


def _row_spec(width):
    return pl.BlockSpec((BM, width), lambda i: (i, 0))


def _full_spec(shape):
    return pl.BlockSpec(shape, lambda i: tuple(0 for _ in shape))


_W = _full_spec((D, D))
_B = _full_spec((1, D))


def _tc1(x, wd, bd, g1, be1):
    return pl.pallas_call(
        _tc1_body,
        grid=(GRID,),
        in_specs=[_row_spec(D), _W, _B, _B, _B],
        out_specs=_row_spec(D),
        out_shape=jax.ShapeDtypeStruct((N, D), jnp.float32),
    )(x, wd, bd, g1, be1)


def _tc2(s, c, h, wl, bl, wr, g, be):
    return pl.pallas_call(
        _tc2_body,
        grid=(GRID,),
        in_specs=[_row_spec(D), _row_spec(CW), _row_spec(D), _W, _B, _W, _B,
                  _B],
        out_specs=_row_spec(D),
        out_shape=jax.ShapeDtypeStruct((N, D), jnp.float32),
    )(s, c, h, wl, bl, wr, g, be)


def _tc3(s, c, h, x, wsc, bsc, wl, bl, wr):
    return pl.pallas_call(
        _tc3_body,
        grid=(GRID,),
        in_specs=[_row_spec(D), _row_spec(CW), _row_spec(D), _row_spec(D),
                  _W, _B, _W, _B, _W],
        out_specs=_row_spec(D),
        out_shape=jax.ShapeDtypeStruct((N, D), jnp.float32),
    )(s, c, h, x, wsc, bsc, wl, bl, wr)


NW = NC * NS
OB = HALF_PAD
OUTP = NC * OB
NV = 2 * NW
TRV = OUTP // NV
SCRAP = TRV
AROWS = TRV + 8
LCAP = 6400
LIMIT = LCAP - K
ECH = 10000
VPB = K // 16


def _sc_mesh():
    return plsc.VectorSubcoreMesh(core_axis_name="c", subcore_axis_name="s")


def _sc_filter_body(src_hbm, dst_hbm, lsrc_hbm, lrow_hbm, lcnt_hbm,
                    esrc_v, edst_v, msrc0_v, mrow0_v, msrc1_v, mrow1_v,
                    cnt_v):
    cid = lax.axis_index("c")
    sid = lax.axis_index("s")
    wid = cid * NS + sid
    lo = wid * 2 * TRV

    def run_chunk(c, carry):
        pltpu.sync_copy(src_hbm.at[pl.ds(c * ECH, ECH)], esrc_v)
        pltpu.sync_copy(dst_hbm.at[pl.ds(c * ECH, ECH)], edst_v)

        def fstep(i, carry):
            off0, off1 = carry
            s = esrc_v[pl.ds(i * 16, 16)]
            d = edst_v[pl.ds(i * 16, 16)]
            p = d + jnp.where(d >= HALF, OB - HALF, 0)
            q = p - lo
            m0 = (q >= 0) & (q < TRV)
            m1 = (q >= TRV) & (q < 2 * TRV)
            plsc.store_compressed(msrc0_v.at[pl.ds(off0, 16)], s, mask=m0)
            plsc.store_compressed(mrow0_v.at[pl.ds(off0, 16)], q, mask=m0)
            plsc.store_compressed(msrc1_v.at[pl.ds(off1, 16)], s, mask=m1)
            plsc.store_compressed(mrow1_v.at[pl.ds(off1, 16)], q - TRV,
                                  mask=m1)
            off0 = jnp.minimum(
                off0 + jnp.max(plsc.all_reduce_population_count(m0)), LIMIT)
            off1 = jnp.minimum(
                off1 + jnp.max(plsc.all_reduce_population_count(m1)), LIMIT)
            return off0, off1

        return plsc.parallel_loop(0, ECH // 16, carry=carry)(fstep)

    cnt0 = jnp.int32(0)
    cnt1 = jnp.int32(0)
    for c in range(E // ECH):
        cnt0, cnt1 = run_chunk(c, (cnt0, cnt1))

    zi16 = jnp.zeros((16,), jnp.int32)
    for j in range(VPB):
        msrc0_v[pl.ds(cnt0 + j * 16, 16)] = zi16 + sid * 625
        mrow0_v[pl.ds(cnt0 + j * 16, 16)] = zi16 + SCRAP
        msrc1_v[pl.ds(cnt1 + j * 16, 16)] = zi16 + sid * 625
        mrow1_v[pl.ds(cnt1 + j * 16, 16)] = zi16 + SCRAP

    cnt_v[pl.ds(0, 16)] = zi16 + cnt0
    cnt_v[pl.ds(16, 16)] = zi16 + cnt1
    v0 = 2 * wid
    pltpu.sync_copy(msrc0_v, lsrc_hbm.at[pl.ds(v0 * LCAP, LCAP)])
    pltpu.sync_copy(mrow0_v, lrow_hbm.at[pl.ds(v0 * LCAP, LCAP)])
    pltpu.sync_copy(msrc1_v, lsrc_hbm.at[pl.ds((v0 + 1) * LCAP, LCAP)])
    pltpu.sync_copy(mrow1_v, lrow_hbm.at[pl.ds((v0 + 1) * LCAP, LCAP)])
    pltpu.sync_copy(cnt_v, lcnt_hbm.at[pl.ds(wid * 32, 32)])


def _sc_filter(src, dst):
    return pl.kernel(
        _sc_filter_body,
        out_type=(
            jax.ShapeDtypeStruct((NV * LCAP,), jnp.int32),
            jax.ShapeDtypeStruct((NV * LCAP,), jnp.int32),
            jax.ShapeDtypeStruct((NW * 32,), jnp.int32),
        ),
        mesh=_sc_mesh(),
        compiler_params=pltpu.CompilerParams(needs_layout_passes=False),
        scratch_types=[
            pltpu.VMEM((ECH,), jnp.int32),
            pltpu.VMEM((ECH,), jnp.int32),
            pltpu.VMEM((LCAP,), jnp.int32),
            pltpu.VMEM((LCAP,), jnp.int32),
            pltpu.VMEM((LCAP,), jnp.int32),
            pltpu.VMEM((LCAP,), jnp.int32),
            pltpu.VMEM((32,), jnp.int32),
        ],
    )(src, dst)


def _sc_agg_body(h_hbm, lsrc_hbm, lrow_hbm, lcnt_hbm, sums_hbm, cnts_hbm,
                 sgidx_v, lsv, lrv, cidx_v, grows_v, acc_s, acc_c,
                 sema, semb):
    cid = lax.axis_index("c")
    sid = lax.axis_index("s")
    wid = cid * NS + sid

    zeros16 = jnp.zeros((16,), jnp.float32)
    ones16 = jnp.ones((16,), jnp.float32)
    pltpu.sync_copy(lcnt_hbm.at[pl.ds(wid * 32, 32)], cidx_v)

    for ph in range(2):
        vid = 2 * wid + ph

        def zrow(i, carry):
            for j in range(D // 16):
                acc_s[i, pl.ds(j * 16, 16)] = zeros16
            acc_c[i, pl.ds(0, 16)] = zeros16
            return carry

        lax.fori_loop(0, AROWS, zrow, 0)

        pltpu.sync_copy(lsrc_hbm.at[pl.ds(vid * LCAP, LCAP)], lsv)
        pltpu.sync_copy(lrow_hbm.at[pl.ds(vid * LCAP, LCAP)], lrv)

        cnt = cidx_v[pl.ds(ph * 16, 16)][0]
        nb = (cnt + (K - 1)) // K

        def start_slot(b, s):
            base = s * K
            for j in range(VPB):
                sgidx_v[pl.ds(base + j * 16, 16)] = (
                    lsv[pl.ds(b * K + j * 16, 16)])
            sem = sema if s == 0 else semb
            pltpu.async_copy(h_hbm.at[sgidx_v.at[pl.ds(base, K)]],
                             grows_v.at[pl.ds(base, K)], sem)

        def wait_slot(s):
            base = s * K
            sem = sema if s == 0 else semb
            pltpu.make_async_copy(h_hbm.at[sgidx_v.at[pl.ds(base, K)]],
                                  grows_v.at[pl.ds(base, K)], sem).wait()

        @pl.when(nb > 0)
        def _():
            start_slot(0, 0)

        def batch(b, carry):
            slot = lax.rem(b, 2)
            more = b + 1 < nb

            @pl.when(more & (slot == 0))
            def _():
                start_slot(b + 1, 1)

            @pl.when(more & (slot == 1))
            def _():
                start_slot(b + 1, 0)

            @pl.when(slot == 0)
            def _():
                wait_slot(0)

            @pl.when(slot == 1)
            def _():
                wait_slot(1)

            gbase = slot * K

            @plsc.parallel_loop(0, VPB)
            def blk(q):
                rv = lrv[pl.ds(b * K + q * 16, 16)]
                for l in range(16):
                    row = rv[l]
                    for j in range(D // 16):
                        plsc.addupdate(acc_s.at[row, pl.ds(j * 16, 16)],
                                       grows_v[gbase + q * 16 + l,
                                               pl.ds(j * 16, 16)])
                    plsc.addupdate(acc_c.at[row, pl.ds(0, 16)], ones16)

            return carry

        lax.fori_loop(0, nb, batch, 0)

        pltpu.sync_copy(acc_s.at[pl.ds(0, TRV)],
                        sums_hbm.at[pl.ds(vid * TRV, TRV)])
        pltpu.sync_copy(acc_c.at[pl.ds(0, TRV)],
                        cnts_hbm.at[pl.ds(vid * TRV, TRV)])


def _sc_agg(h, lsrc, lrow, lcnt):
    return pl.kernel(
        _sc_agg_body,
        out_type=(
            jax.ShapeDtypeStruct((OUTP, D), jnp.float32),
            jax.ShapeDtypeStruct((OUTP, CW), jnp.float32),
        ),
        mesh=_sc_mesh(),
        scratch_types=[
            pltpu.VMEM((2 * K,), jnp.int32),
            pltpu.VMEM((LCAP,), jnp.int32),
            pltpu.VMEM((LCAP,), jnp.int32),
            pltpu.VMEM((32,), jnp.int32),
            pltpu.VMEM((2 * K, D), jnp.float32),
            pltpu.VMEM((AROWS, D), jnp.float32),
            pltpu.VMEM((AROWS, CW), jnp.float32),
            pltpu.SemaphoreType.DMA,
            pltpu.SemaphoreType.DMA,
        ],
    )(h, lsrc, lrow, lcnt)


def _unpad(a):
    return a.reshape(NC, OB, a.shape[-1])[:, :HALF].reshape(N, a.shape[-1])


def kernel(x, edges, W_down, b_down, W_sc, b_sc, g1, be1, Wl1, bl1, Wr1,
           g2, be2, Wl2, bl2, Wr2):
    src = edges[0]
    dst = edges[1]
    r = lambda v: v.reshape(1, D)
    lsrc, lrow, lcnt = _sc_filter(src, dst)
    h0 = _tc1(x, W_down, r(b_down), r(g1), r(be1))
    s1f, cntsf = _sc_agg(h0, lsrc, lrow, lcnt)
    s1, cnts = _unpad(s1f), _unpad(cntsf)
    g1p = _tc2(s1, cnts, h0, Wl1, r(bl1), Wr1, r(g2), r(be2))
    s2f, _ = _sc_agg(g1p, lsrc, lrow, lcnt)
    s2 = _unpad(s2f)
    return _tc3(s2, cnts, g1p, x, W_sc, r(b_sc), Wl2, r(bl2), Wr2)
